# baseline (speedup 1.0000x reference)
import functools

import jax
import jax.numpy as jnp
from jax import lax
from jax.experimental import pallas as pl
from jax.experimental.pallas import tpu as pltpu
from jax.experimental.pallas import tpu_sc as plsc

NC, NS = 2, 16
NW = NC * NS
IDX_W = 128


def _sc_embed(in_flat, table_flat, postab, offtab, *, ch, n_dma):
    mesh = plsc.VectorSubcoreMesh(
        core_axis_name="c", subcore_axis_name="s",
        num_cores=NC, num_subcores=NS)

    @functools.partial(
        pl.kernel,
        out_type=jax.ShapeDtypeStruct(in_flat.shape, jnp.float32),
        mesh=mesh,
        compiler_params=pltpu.CompilerParams(needs_layout_passes=False),
        scratch_types=[
            pltpu.VMEM((ch,), jnp.float32),
            pltpu.VMEM((n_dma, IDX_W), jnp.int32),
            pltpu.VMEM((n_dma, IDX_W), jnp.float32),
            pltpu.VMEM((n_dma, IDX_W), jnp.int32),
            pltpu.VMEM((n_dma, IDX_W), jnp.int32),
            pltpu.SemaphoreType.DMA,
        ],
    )
    def k(in_hbm, tab_hbm, pos_hbm, off_hbm, out_hbm,
          chunkv, idxv, valsv, posv, offv, gsem):
        wid = lax.axis_index("s") * NC + lax.axis_index("c")
        base = wid * ch
        pltpu.sync_copy(in_hbm.at[pl.ds(base, ch)], chunkv)
        pltpu.sync_copy(pos_hbm, posv)
        pltpu.sync_copy(off_hbm, offv)

        @pl.loop(0, n_dma)
        def _fire(j):
            for i in range(IDX_W // 16):
                pv = posv[j, pl.ds(i * 16, 16)]
                raw = plsc.load_gather(chunkv, [pv])
                idxv[j, pl.ds(i * 16, 16)] = (
                    raw.astype(jnp.int32) + offv[j, pl.ds(i * 16, 16)])
            pltpu.async_copy(tab_hbm.at[idxv.at[j]], valsv.at[j], gsem)

        @pl.loop(0, n_dma)
        def _drain(j):
            pltpu.make_async_copy(
                tab_hbm.at[idxv.at[0]], valsv.at[0], gsem).wait()

        @pl.loop(0, n_dma)
        def _place(j):
            for i in range(IDX_W // 16):
                pv = posv[j, pl.ds(i * 16, 16)]
                vv = valsv[j, pl.ds(i * 16, 16)]
                plsc.store_scatter(chunkv, [pv], vv)

        pltpu.sync_copy(chunkv, out_hbm.at[pl.ds(base, ch)])

    return k(in_flat, table_flat, postab, offtab)


def kernel(input, table, embedding_indices, offsets):
    B, F = input.shape
    E = embedding_indices.shape[0]
    rows_per_worker = B // NW
    ch = rows_per_worker * F
    pk = rows_per_worker * E
    n_dma = pk // IDX_W

    q = jnp.arange(pk, dtype=jnp.int32)
    row, col = q // E, q % E
    postab = (row * F + embedding_indices[col]).astype(jnp.int32)
    offtab = offsets[col].astype(jnp.int32)

    out_flat = _sc_embed(
        input.reshape(-1), table.reshape(-1),
        postab.reshape(n_dma, IDX_W), offtab.reshape(n_dma, IDX_W),
        ch=ch, n_dma=n_dma)
    return out_flat.reshape(B, F)


# speedup vs baseline: 1.2723x; 1.2723x over previous
import functools

import jax
import jax.numpy as jnp
from jax import lax
from jax.experimental import pallas as pl
from jax.experimental.pallas import tpu as pltpu
from jax.experimental.pallas import tpu_sc as plsc

NC, NS = 2, 16
NW = NC * NS
IDX_W = 128


def _sc_embed(in_flat, table_flat, emb_idx, offsets, *, ch, n_dma, E, F):
    mesh = plsc.VectorSubcoreMesh(
        core_axis_name="c", subcore_axis_name="s",
        num_cores=NC, num_subcores=NS)

    @functools.partial(
        pl.kernel,
        out_type=jax.ShapeDtypeStruct(in_flat.shape, jnp.float32),
        mesh=mesh,
        compiler_params=pltpu.CompilerParams(needs_layout_passes=False),
        scratch_types=[
            pltpu.VMEM((ch,), jnp.float32),
            pltpu.VMEM((n_dma, IDX_W), jnp.int32),
            pltpu.VMEM((n_dma, IDX_W), jnp.float32),
            pltpu.VMEM((n_dma, IDX_W), jnp.int32),
            pltpu.VMEM((E,), jnp.int32),
            pltpu.VMEM((E,), jnp.int32),
            pltpu.SemaphoreType.DMA,
        ],
    )
    def k(in_hbm, tab_hbm, emb_hbm, off_hbm, out_hbm,
          chunkv, idxv, valsv, posv, embv, offv, gsem):
        wid = lax.axis_index("s") * NC + lax.axis_index("c")
        base = wid * ch
        pltpu.sync_copy(emb_hbm, embv)
        pltpu.sync_copy(off_hbm, offv)
        pltpu.sync_copy(in_hbm.at[pl.ds(base, ch)], chunkv)

        lanes = lax.iota(jnp.int32, 16)

        @pl.loop(0, n_dma)
        def _fire(j):
            for i in range(IDX_W // 16):
                qv = j * IDX_W + i * 16 + lanes
                rowv = qv // E
                colv = qv - rowv * E
                pv = rowv * F + plsc.load_gather(embv, [colv])
                posv[j, pl.ds(i * 16, 16)] = pv
                raw = plsc.load_gather(chunkv, [pv])
                idxv[j, pl.ds(i * 16, 16)] = (
                    raw.astype(jnp.int32) + plsc.load_gather(offv, [colv]))
            pltpu.async_copy(tab_hbm.at[idxv.at[j]], valsv.at[j], gsem)

        @pl.loop(0, n_dma)
        def _drain(j):
            pltpu.make_async_copy(
                tab_hbm.at[idxv.at[0]], valsv.at[0], gsem).wait()

        @pl.loop(0, n_dma)
        def _place(j):
            for i in range(IDX_W // 16):
                pv = posv[j, pl.ds(i * 16, 16)]
                vv = valsv[j, pl.ds(i * 16, 16)]
                plsc.store_scatter(chunkv, [pv], vv)

        pltpu.sync_copy(chunkv, out_hbm.at[pl.ds(base, ch)])

    return k(in_flat, table_flat, emb_idx, offsets)


def kernel(input, table, embedding_indices, offsets):
    B, F = input.shape
    E = embedding_indices.shape[0]
    rows_per_worker = B // NW
    ch = rows_per_worker * F
    pk = rows_per_worker * E
    n_dma = pk // IDX_W

    out_flat = _sc_embed(
        input.reshape(-1), table.reshape(-1),
        embedding_indices, offsets,
        ch=ch, n_dma=n_dma, E=E, F=F)
    return out_flat.reshape(B, F)
